# baseline (speedup 1.0000x reference)
import functools

import jax
import jax.numpy as jnp
from jax import lax
from jax.experimental import pallas as pl
from jax.experimental.pallas import tpu as pltpu
from jax.experimental.pallas import tpu_sc as plsc

_NUM_FIELDS = 26
_FIELD_DIM = 40000
_BATCH = 16384
_EMB = 16
_NUM_ROWS = _NUM_FIELDS * _FIELD_DIM
_TOTAL = _BATCH * _NUM_FIELDS
_NW = 32
_PER_W = _TOTAL // _NW
_IDXW = 128
_BLKS = 8
_CHUNK = _BLKS * _IDXW
_NCHUNK = _PER_W // _CHUNK
_TOTAL_BLKS = _TOTAL // _IDXW
_W_BLKS = _PER_W // _IDXW
_TCOLS = _NUM_ROWS // _IDXW
_TC_PER_W = -(-_TCOLS // _NW)

_MESH = dict(core_axis_name="c", subcore_axis_name="s")


_GW = 8
_NGRP = _TCOLS // _GW
_GRP_PER_W = -(-_NGRP // _NW)
_GHALF = _GW * 1024
_GVOUT = _GW * _IDXW * _EMB


def _sc_detile(t2):
  mesh = plsc.VectorSubcoreMesh(**_MESH)

  @functools.partial(
      pl.kernel,
      mesh=mesh,
      compiler_params=pltpu.CompilerParams(
          use_tc_tiling_on_sc=False, needs_layout_passes=False),
      out_type=jax.ShapeDtypeStruct((_NUM_ROWS * _EMB,), jnp.float32),
      scratch_types=[
          pltpu.VMEM((2, 2 * _GHALF), jnp.float32),
          pltpu.VMEM((2, _GVOUT), jnp.float32),
          pltpu.SemaphoreType.DMA,
          pltpu.SemaphoreType.DMA,
          pltpu.SemaphoreType.DMA,
          pltpu.SemaphoreType.DMA,
      ],
  )
  def k(t2_hbm, lin_hbm, vin, vout, i0, i1, o0, o1):
    wid = lax.axis_index("s") * 2 + lax.axis_index("c")
    isem = (i0, i1)
    osem = (o0, o1)
    lane = lax.iota(jnp.int32, 16)
    biota = ((lane >> 3) * _GHALF) + ((lane & 7) << 7)

    def gid_of(g):
      return wid * _GRP_PER_W + g

    def fire_in(g, slot):
      gid = gid_of(g)
      for r in range(2):
        pltpu.async_copy(
            t2_hbm.at[r, pl.ds(gid * _GHALF, _GHALF)],
            vin.at[slot, pl.ds(r * _GHALF, _GHALF)], isem[slot])

    def drain_in(g, slot):
      gid = gid_of(g)
      for r in range(2):
        pltpu.make_async_copy(
            t2_hbm.at[r, pl.ds(gid * _GHALF, _GHALF)],
            vin.at[slot, pl.ds(r * _GHALF, _GHALF)], isem[slot]).wait()

    def out_slice(g):
      return lin_hbm.at[pl.ds(gid_of(g) * _GVOUT, _GVOUT)]

    fire_in(0, 0)
    for g in range(_GRP_PER_W):
      slot = g & 1
      valid = gid_of(g) < _NGRP
      if g + 1 < _GRP_PER_W:
        @pl.when(gid_of(g + 1) < _NGRP)
        def _(g=g, slot=slot):
          fire_in(g + 1, 1 - slot)
      if g >= 2:
        @pl.when(gid_of(g - 2) < _NGRP)
        def _(g=g, slot=slot):
          pltpu.make_async_copy(vout.at[slot], out_slice(g - 2),
                                osem[slot]).wait()

      @pl.when(valid)
      def _(g=g, slot=slot):
        drain_in(g, slot)

        def tr_body(l, carry):
          s16 = l * 16
          for wi in range(_GW):
            idx = biota + (l + wi * 1024)
            vec = plsc.load_gather(vin.at[slot], [idx])
            vout[slot, pl.ds(s16 + wi * 2048, 16)] = vec
          return carry

        lax.fori_loop(0, _IDXW, tr_body, 0, unroll=2)
        pltpu.async_copy(vout.at[slot], out_slice(g), osem[slot])

    for g in (_GRP_PER_W - 2, _GRP_PER_W - 1):
      @pl.when(gid_of(g) < _NGRP)
      def _(g=g):
        pltpu.make_async_copy(vout.at[g & 1], out_slice(g),
                              osem[g & 1]).wait()

    @pl.when(wid < _TCOLS - _NGRP * _GW)
    def _():
      c = _NGRP * _GW + wid
      for r in range(2):
        pltpu.sync_copy(t2_hbm.at[r, pl.ds(c * 1024, 1024)],
                        vin.at[0, pl.ds(r * _GHALF, 1024)])

      def tail_body(l, carry):
        vec = plsc.load_gather(vin.at[0], [biota + l])
        vout[0, pl.ds(l * 16, 16)] = vec
        return carry

      lax.fori_loop(0, _IDXW, tail_body, 0)
      pltpu.sync_copy(vout.at[0, pl.ds(0, _IDXW * _EMB)],
                      lin_hbm.at[pl.ds(c * _IDXW * _EMB, _IDXW * _EMB)])

  return k(t2)


def _sc_gather(x_fm, lin_table):
  mesh = plsc.VectorSubcoreMesh(**_MESH)

  @functools.partial(
      pl.kernel,
      mesh=mesh,
      compiler_params=pltpu.CompilerParams(use_tc_tiling_on_sc=False),
      out_type=jax.ShapeDtypeStruct((_TOTAL_BLKS, _IDXW, _EMB), jnp.float32),
      scratch_types=[
          pltpu.VMEM((_PER_W,), jnp.int32),
          pltpu.VMEM((2, _BLKS, _IDXW, _EMB), jnp.float32),
          pltpu.SemaphoreType.DMA,
          pltpu.SemaphoreType.DMA,
          pltpu.SemaphoreType.DMA,
          pltpu.SemaphoreType.DMA,
      ],
  )
  def k(x_hbm, table_hbm, out_hbm, idx_v, rows_v, g0, g1, o0, o1):
    wid = lax.axis_index("s") * 2 + lax.axis_index("c")
    base = wid * _PER_W
    wblk = wid * _W_BLKS
    gsem = (g0, g1)
    osem = (o0, o1)

    pltpu.sync_copy(x_hbm.at[pl.ds(base, _PER_W)], idx_v)

    def out_slice(c):
      return out_hbm.at[pl.ds(wblk + c * _BLKS, _BLKS)]

    prev = None
    for c in range(_NCHUNK):
      slot = c & 1
      off = ((base + c * _CHUNK) >> 14) * _FIELD_DIM

      def add_body(i, carry, c=c, off=off):
        s = pl.ds(c * _CHUNK + i * 16, 16)
        idx_v[s] = idx_v[s] + off
        return carry

      lax.fori_loop(0, _CHUNK // 16, add_body, 0)

      if c >= 2:
        pltpu.make_async_copy(rows_v.at[slot], out_slice(c - 2),
                              osem[slot]).wait()
      hs = [
          pltpu.async_copy(
              table_hbm.at[idx_v.at[pl.ds(c * _CHUNK + j * _IDXW, _IDXW)]],
              rows_v.at[slot, j], gsem[slot])
          for j in range(_BLKS)
      ]
      if prev is not None:
        for h in prev:
          h.wait()
        pltpu.async_copy(rows_v.at[1 - slot], out_slice(c - 1),
                         osem[1 - slot])
      prev = hs

    last = _NCHUNK - 1
    for h in prev:
      h.wait()
    pltpu.async_copy(rows_v.at[last & 1], out_slice(last), osem[last & 1])
    pltpu.make_async_copy(rows_v.at[(last - 1) & 1], out_slice(last - 1),
                          osem[(last - 1) & 1]).wait()
    pltpu.make_async_copy(rows_v.at[last & 1], out_slice(last),
                          osem[last & 1]).wait()

  return k(x_fm, lin_table)


def kernel(x, table):
  t2 = table.T.reshape(2, 8, _TCOLS, _IDXW).transpose(0, 2, 1, 3).reshape(
      2, _TCOLS * 8 * _IDXW)
  lin = _sc_detile(t2).reshape(_NUM_ROWS, _EMB)
  x_fm = x.T.reshape(_TOTAL).astype(jnp.int32)
  out = _sc_gather(x_fm, lin)
  return out.reshape(_NUM_FIELDS, _BATCH, _EMB).transpose(1, 0, 2)


# speedup vs baseline: 1.2156x; 1.2156x over previous
import functools

import jax
import jax.numpy as jnp
from jax import lax
from jax.experimental import pallas as pl
from jax.experimental.pallas import tpu as pltpu
from jax.experimental.pallas import tpu_sc as plsc

_NUM_FIELDS = 26
_FIELD_DIM = 40000
_BATCH = 16384
_EMB = 16
_NUM_ROWS = _NUM_FIELDS * _FIELD_DIM
_TOTAL = _BATCH * _NUM_FIELDS
_NW = 32
_PER_W = _TOTAL // _NW
_IDXW = 128
_BLKS = 8
_CHUNK = _BLKS * _IDXW
_NCHUNK = _PER_W // _CHUNK
_TOTAL_BLKS = _TOTAL // _IDXW
_W_BLKS = _PER_W // _IDXW
_TCOLS = _NUM_ROWS // _IDXW
_TC_PER_W = -(-_TCOLS // _NW)

_MESH = dict(core_axis_name="c", subcore_axis_name="s")


_GW = 8
_NGRP = _TCOLS // _GW
_GRP_PER_W = -(-_NGRP // _NW)
_GHALF = _GW * 1024
_GVOUT = _GW * _IDXW * _EMB


def _sc_detile(t2):
  mesh = plsc.VectorSubcoreMesh(**_MESH)

  @functools.partial(
      pl.kernel,
      mesh=mesh,
      compiler_params=pltpu.CompilerParams(
          use_tc_tiling_on_sc=False, needs_layout_passes=False),
      out_type=jax.ShapeDtypeStruct((_NUM_ROWS, _EMB), jnp.float32),
      scratch_types=[
          pltpu.VMEM((2, 2 * _GHALF), jnp.float32),
          pltpu.VMEM((2, _GW * _IDXW, 17), jnp.float32),
          pltpu.SemaphoreType.DMA,
          pltpu.SemaphoreType.DMA,
          pltpu.SemaphoreType.DMA,
          pltpu.SemaphoreType.DMA,
      ],
  )
  def k(t2_hbm, lin_hbm, vin, vout, i0, i1, o0, o1):
    wid = lax.axis_index("s") * 2 + lax.axis_index("c")
    isem = (i0, i1)
    osem = (o0, o1)
    lane = lax.iota(jnp.int32, 16)

    def gid_of(g):
      return wid * _GRP_PER_W + g

    def fire_in(g, slot):
      gid = gid_of(g)
      for r in range(2):
        pltpu.async_copy(
            t2_hbm.at[r, pl.ds(gid * _GHALF, _GHALF)],
            vin.at[slot, pl.ds(r * _GHALF, _GHALF)], isem[slot])

    def drain_in(g, slot):
      gid = gid_of(g)
      for r in range(2):
        pltpu.make_async_copy(
            t2_hbm.at[r, pl.ds(gid * _GHALF, _GHALF)],
            vin.at[slot, pl.ds(r * _GHALF, _GHALF)], isem[slot]).wait()

    def out_slice(g):
      return lin_hbm.at[pl.ds(gid_of(g) * _GW * _IDXW, _GW * _IDXW)]

    def vout_data(slot):
      return vout.at[slot, :, pl.ds(0, _EMB)]

    fire_in(0, 0)
    for g in range(_GRP_PER_W):
      slot = g & 1
      valid = gid_of(g) < _NGRP
      if g + 1 < _GRP_PER_W:
        @pl.when(gid_of(g + 1) < _NGRP)
        def _(g=g, slot=slot):
          fire_in(g + 1, 1 - slot)
      if g >= 2:
        @pl.when(gid_of(g - 2) < _NGRP)
        def _(g=g, slot=slot):
          pltpu.make_async_copy(vout_data(slot), out_slice(g - 2),
                                osem[slot]).wait()

      @pl.when(valid)
      def _(g=g, slot=slot):
        drain_in(g, slot)

        def tr_body(t, carry):
          wi = t >> 3
          lb = t & 7
          a0 = wi * 1024 + lb * 16
          rowv = (wi * _IDXW + lb * 16) + lane
          for e in range(16):
            vec = vin[slot, pl.ds(a0 + (e >> 3) * _GHALF + (e & 7) * 128, 16)]
            col = jnp.full((16,), e, jnp.int32)
            plsc.store_scatter(vout.at[slot], [rowv, col], vec)
          return carry

        lax.fori_loop(0, _GW * 8, tr_body, 0)
        pltpu.async_copy(vout_data(slot), out_slice(g), osem[slot])

    for g in (_GRP_PER_W - 2, _GRP_PER_W - 1):
      @pl.when(gid_of(g) < _NGRP)
      def _(g=g):
        pltpu.make_async_copy(vout_data(g & 1), out_slice(g),
                              osem[g & 1]).wait()

    @pl.when(wid < _TCOLS - _NGRP * _GW)
    def _():
      c = _NGRP * _GW + wid
      for r in range(2):
        pltpu.sync_copy(t2_hbm.at[r, pl.ds(c * 1024, 1024)],
                        vin.at[0, pl.ds(r * _GHALF, 1024)])

      def tail_body(t, carry):
        rowv = t * 16 + lane
        for e in range(16):
          vec = vin[0, pl.ds(t * 16 + (e >> 3) * _GHALF + (e & 7) * 128, 16)]
          col = jnp.full((16,), e, jnp.int32)
          plsc.store_scatter(vout.at[0], [rowv, col], vec)
        return carry

      lax.fori_loop(0, 8, tail_body, 0)
      pltpu.sync_copy(vout.at[0, pl.ds(0, _IDXW), pl.ds(0, _EMB)],
                      lin_hbm.at[pl.ds(c * _IDXW, _IDXW)])

  return k(t2)


def _sc_gather(x_fm, lin_table):
  mesh = plsc.VectorSubcoreMesh(**_MESH)

  @functools.partial(
      pl.kernel,
      mesh=mesh,
      compiler_params=pltpu.CompilerParams(use_tc_tiling_on_sc=False),
      out_type=jax.ShapeDtypeStruct((_TOTAL_BLKS, _IDXW, _EMB), jnp.float32),
      scratch_types=[
          pltpu.VMEM((_PER_W,), jnp.int32),
          pltpu.VMEM((2, _BLKS, _IDXW, _EMB), jnp.float32),
          pltpu.SemaphoreType.DMA,
          pltpu.SemaphoreType.DMA,
          pltpu.SemaphoreType.DMA,
          pltpu.SemaphoreType.DMA,
      ],
  )
  def k(x_hbm, table_hbm, out_hbm, idx_v, rows_v, g0, g1, o0, o1):
    wid = lax.axis_index("s") * 2 + lax.axis_index("c")
    base = wid * _PER_W
    wblk = wid * _W_BLKS
    gsem = (g0, g1)
    osem = (o0, o1)

    pltpu.sync_copy(x_hbm.at[pl.ds(base, _PER_W)], idx_v)

    def out_slice(c):
      return out_hbm.at[pl.ds(wblk + c * _BLKS, _BLKS)]

    prev = None
    for c in range(_NCHUNK):
      slot = c & 1
      off = ((base + c * _CHUNK) >> 14) * _FIELD_DIM

      def add_body(i, carry, c=c, off=off):
        s = pl.ds(c * _CHUNK + i * 16, 16)
        idx_v[s] = idx_v[s] + off
        return carry

      lax.fori_loop(0, _CHUNK // 16, add_body, 0)

      if c >= 2:
        pltpu.make_async_copy(rows_v.at[slot], out_slice(c - 2),
                              osem[slot]).wait()
      hs = [
          pltpu.async_copy(
              table_hbm.at[idx_v.at[pl.ds(c * _CHUNK + j * _IDXW, _IDXW)]],
              rows_v.at[slot, j], gsem[slot])
          for j in range(_BLKS)
      ]
      if prev is not None:
        for h in prev:
          h.wait()
        pltpu.async_copy(rows_v.at[1 - slot], out_slice(c - 1),
                         osem[1 - slot])
      prev = hs

    last = _NCHUNK - 1
    for h in prev:
      h.wait()
    pltpu.async_copy(rows_v.at[last & 1], out_slice(last), osem[last & 1])
    pltpu.make_async_copy(rows_v.at[(last - 1) & 1], out_slice(last - 1),
                          osem[(last - 1) & 1]).wait()
    pltpu.make_async_copy(rows_v.at[last & 1], out_slice(last),
                          osem[last & 1]).wait()

  return k(x_fm, lin_table)


def kernel(x, table):
  t2 = table.T.reshape(2, 8, _TCOLS, _IDXW).transpose(0, 2, 1, 3).reshape(
      2, _TCOLS * 8 * _IDXW)
  lin = _sc_detile(t2)
  x_fm = x.T.reshape(_TOTAL).astype(jnp.int32)
  out = _sc_gather(x_fm, lin)
  return out.reshape(_NUM_FIELDS, _BATCH, _EMB).transpose(1, 0, 2)


# speedup vs baseline: 1.2167x; 1.0009x over previous
import functools

import jax
import jax.numpy as jnp
from jax import lax
from jax.experimental import pallas as pl
from jax.experimental.pallas import tpu as pltpu
from jax.experimental.pallas import tpu_sc as plsc

_NUM_FIELDS = 26
_FIELD_DIM = 40000
_BATCH = 16384
_EMB = 16
_NUM_ROWS = _NUM_FIELDS * _FIELD_DIM
_TOTAL = _BATCH * _NUM_FIELDS
_NW = 32
_PER_W = _TOTAL // _NW
_IDXW = 128
_BLKS = 8
_CHUNK = _BLKS * _IDXW
_NCHUNK = _PER_W // _CHUNK
_TOTAL_BLKS = _TOTAL // _IDXW
_W_BLKS = _PER_W // _IDXW
_TCOLS = _NUM_ROWS // _IDXW
_TC_PER_W = -(-_TCOLS // _NW)

_MESH = dict(core_axis_name="c", subcore_axis_name="s")


_GW = 8
_NGRP = _TCOLS // _GW
_GRP_PER_W = -(-_NGRP // _NW)
_GHALF = _GW * 1024
_GVOUT = _GW * _IDXW * _EMB


def _sc_detile(t2):
  mesh = plsc.VectorSubcoreMesh(**_MESH)

  @functools.partial(
      pl.kernel,
      mesh=mesh,
      compiler_params=pltpu.CompilerParams(
          use_tc_tiling_on_sc=False, needs_layout_passes=False),
      out_type=jax.ShapeDtypeStruct((_NUM_ROWS, _EMB), jnp.float32),
      scratch_types=[
          pltpu.VMEM((2, 2 * _GHALF), jnp.float32),
          pltpu.VMEM((2, _GW * _IDXW, 17), jnp.float32),
          pltpu.SemaphoreType.DMA,
          pltpu.SemaphoreType.DMA,
          pltpu.SemaphoreType.DMA,
          pltpu.SemaphoreType.DMA,
      ],
  )
  def k(t2_hbm, lin_hbm, vin, vout, i0, i1, o0, o1):
    wid = lax.axis_index("s") * 2 + lax.axis_index("c")
    isem = (i0, i1)
    osem = (o0, o1)
    lane = lax.iota(jnp.int32, 16)

    def gid_of(g):
      return wid * _GRP_PER_W + g

    def fire_in(g, slot):
      gid = gid_of(g)
      for r in range(2):
        pltpu.async_copy(
            t2_hbm.at[r, pl.ds(gid * _GHALF, _GHALF)],
            vin.at[slot, pl.ds(r * _GHALF, _GHALF)], isem[slot])

    def drain_in(g, slot):
      gid = gid_of(g)
      for r in range(2):
        pltpu.make_async_copy(
            t2_hbm.at[r, pl.ds(gid * _GHALF, _GHALF)],
            vin.at[slot, pl.ds(r * _GHALF, _GHALF)], isem[slot]).wait()

    def out_slice(g):
      return lin_hbm.at[pl.ds(gid_of(g) * _GW * _IDXW, _GW * _IDXW)]

    def vout_data(slot):
      return vout.at[slot, :, pl.ds(0, _EMB)]

    fire_in(0, 0)
    for g in range(_GRP_PER_W):
      slot = g & 1
      valid = gid_of(g) < _NGRP
      if g + 1 < _GRP_PER_W:
        @pl.when(gid_of(g + 1) < _NGRP)
        def _(g=g, slot=slot):
          fire_in(g + 1, 1 - slot)
      if g >= 2:
        @pl.when(gid_of(g - 2) < _NGRP)
        def _(g=g, slot=slot):
          pltpu.make_async_copy(vout_data(slot), out_slice(g - 2),
                                osem[slot]).wait()

      @pl.when(valid)
      def _(g=g, slot=slot):
        drain_in(g, slot)

        def tr_body(t, carry):
          wi = t >> 3
          lb = t & 7
          a0 = wi * 1024 + lb * 16
          rowv = (wi * _IDXW + lb * 16) + lane
          for e in range(16):
            vec = vin[slot, pl.ds(a0 + (e >> 3) * _GHALF + (e & 7) * 128, 16)]
            col = jnp.full((16,), e, jnp.int32)
            plsc.store_scatter(vout.at[slot], [rowv, col], vec)
          return carry

        lax.fori_loop(0, _GW * 8, tr_body, 0)
        pltpu.async_copy(vout_data(slot), out_slice(g), osem[slot])

    for g in (_GRP_PER_W - 2, _GRP_PER_W - 1):
      @pl.when(gid_of(g) < _NGRP)
      def _(g=g):
        pltpu.make_async_copy(vout_data(g & 1), out_slice(g),
                              osem[g & 1]).wait()

    @pl.when(wid < _TCOLS - _NGRP * _GW)
    def _():
      c = _NGRP * _GW + wid
      for r in range(2):
        pltpu.sync_copy(t2_hbm.at[r, pl.ds(c * 1024, 1024)],
                        vin.at[0, pl.ds(r * _GHALF, 1024)])

      def tail_body(t, carry):
        rowv = t * 16 + lane
        for e in range(16):
          vec = vin[0, pl.ds(t * 16 + (e >> 3) * _GHALF + (e & 7) * 128, 16)]
          col = jnp.full((16,), e, jnp.int32)
          plsc.store_scatter(vout.at[0], [rowv, col], vec)
        return carry

      lax.fori_loop(0, 8, tail_body, 0)
      pltpu.sync_copy(vout.at[0, pl.ds(0, _IDXW), pl.ds(0, _EMB)],
                      lin_hbm.at[pl.ds(c * _IDXW, _IDXW)])

  return k(t2)


def _sc_gather(x_fm, lin_table):
  mesh = plsc.VectorSubcoreMesh(**_MESH)

  @functools.partial(
      pl.kernel,
      mesh=mesh,
      compiler_params=pltpu.CompilerParams(use_tc_tiling_on_sc=False),
      out_type=jax.ShapeDtypeStruct(
          (_NUM_FIELDS, _BATCH // _IDXW, _IDXW, _EMB), jnp.float32),
      scratch_types=[
          pltpu.VMEM((_PER_W,), jnp.int32),
          pltpu.VMEM((2, _BLKS, _IDXW, _EMB), jnp.float32),
          pltpu.SemaphoreType.DMA,
          pltpu.SemaphoreType.DMA,
          pltpu.SemaphoreType.DMA,
          pltpu.SemaphoreType.DMA,
      ],
  )
  def k(x_hbm, table_hbm, out_hbm, idx_v, rows_v, g0, g1, o0, o1):
    wid = lax.axis_index("s") * 2 + lax.axis_index("c")
    base = wid * _PER_W
    gsem = (g0, g1)
    osem = (o0, o1)

    pltpu.sync_copy(x_hbm.at[pl.ds(base, _PER_W)], idx_v)

    def out_slice(c):
      pos = base + c * _CHUNK
      return out_hbm.at[pos >> 14, pl.ds((pos >> 7) & 127, _BLKS)]

    prev = None
    for c in range(_NCHUNK):
      slot = c & 1
      off = ((base + c * _CHUNK) >> 14) * _FIELD_DIM

      def add_body(i, carry, c=c, off=off):
        s = pl.ds(c * _CHUNK + i * 16, 16)
        idx_v[s] = idx_v[s] + off
        return carry

      lax.fori_loop(0, _CHUNK // 16, add_body, 0)

      if c >= 2:
        pltpu.make_async_copy(rows_v.at[slot], out_slice(c - 2),
                              osem[slot]).wait()
      hs = [
          pltpu.async_copy(
              table_hbm.at[idx_v.at[pl.ds(c * _CHUNK + j * _IDXW, _IDXW)]],
              rows_v.at[slot, j], gsem[slot])
          for j in range(_BLKS)
      ]
      if prev is not None:
        for h in prev:
          h.wait()
        pltpu.async_copy(rows_v.at[1 - slot], out_slice(c - 1),
                         osem[1 - slot])
      prev = hs

    last = _NCHUNK - 1
    for h in prev:
      h.wait()
    pltpu.async_copy(rows_v.at[last & 1], out_slice(last), osem[last & 1])
    pltpu.make_async_copy(rows_v.at[(last - 1) & 1], out_slice(last - 1),
                          osem[(last - 1) & 1]).wait()
    pltpu.make_async_copy(rows_v.at[last & 1], out_slice(last),
                          osem[last & 1]).wait()

  return k(x_fm, lin_table)


def kernel(x, table):
  t2 = table.T.reshape(2, 8, _TCOLS, _IDXW).transpose(0, 2, 1, 3).reshape(
      2, _TCOLS * 8 * _IDXW)
  lin = _sc_detile(t2)
  x_fm = x.T.reshape(_TOTAL).astype(jnp.int32)
  out = _sc_gather(x_fm, lin)
  return out.reshape(_NUM_FIELDS, _BATCH, _EMB).transpose(1, 0, 2)
